# Initial kernel scaffold; baseline (speedup 1.0000x reference)
import jax
import jax.numpy as jnp
from jax.experimental import pallas as pl


def kernel(previous_encoded_m, sim_weights):
    raise NotImplementedError("write your pallas kernel here")


# speedup vs baseline: 6.7777x; 6.7777x over previous
import functools

import jax
import jax.numpy as jnp
from jax import lax
from jax.experimental import pallas as pl
from jax.experimental.pallas import tpu as pltpu
from jax.experimental.pallas import tpu_sc as plsc

_L = 16


def _select_body(B, N, D, rows_per_worker, num_workers,
                 sw_hbm, mem_hbm, out_hbm, w_v, row_v):
  cid = lax.axis_index("c")
  sid = lax.axis_index("s")
  wid = sid * 2 + cid

  lane = lax.iota(jnp.int32, _L)

  for r in range(rows_per_worker):
    b = wid * rows_per_worker + r

    pltpu.sync_copy(sw_hbm.at[b], w_v)

    def step(j, carry):
      bv, bi = carry
      vals = w_v[pl.ds(j * _L, _L)]
      m = vals > bv
      bv = jnp.where(m, vals, bv)
      bi = jnp.where(m, j * _L + lane, bi)
      return bv, bi

    best_v, best_i = lax.fori_loop(
        1, N // _L, step, (w_v[pl.ds(0, _L)], lane))

    bv, bi = best_v, best_i
    for k in (8, 4, 2, 1):
      perm = lane ^ k
      ov = bv.at[perm].get(mode="promise_in_bounds")
      oi = bi.at[perm].get(mode="promise_in_bounds")
      better = (ov > bv) | ((ov == bv) & (oi < bi))
      bv = jnp.where(better, ov, bv)
      bi = jnp.where(better, oi, bi)
    vmax = bv[0]
    imax = bi[0]
    scale = vmax + (jnp.float32(1.0) - vmax)

    pltpu.sync_copy(mem_hbm.at[b * N + imax], row_v)

    def smul(j, _):
      row_v[pl.ds(j * _L, _L)] = row_v[pl.ds(j * _L, _L)] * scale
      return 0

    lax.fori_loop(0, D // _L, smul, 0)

    pltpu.sync_copy(row_v, out_hbm.at[b])


def kernel(previous_encoded_m, sim_weights):
  B, N = sim_weights.shape
  D = previous_encoded_m.shape[2]
  num_workers = 32
  rows_per_worker = B // num_workers

  table = previous_encoded_m.reshape(B * N, D)

  mesh = plsc.VectorSubcoreMesh(core_axis_name="c", subcore_axis_name="s")
  body = functools.partial(_select_body, B, N, D, rows_per_worker,
                           num_workers)
  run = pl.kernel(
      body,
      mesh=mesh,
      out_type=jax.ShapeDtypeStruct((B, D), jnp.float32),
      scratch_types=[
          pltpu.VMEM((N,), jnp.float32),
          pltpu.VMEM((D,), jnp.float32),
      ],
  )
  return run(sim_weights, table)


# speedup vs baseline: 7.3466x; 1.0839x over previous
import functools

import jax
import jax.numpy as jnp
from jax import lax
from jax.experimental import pallas as pl
from jax.experimental.pallas import tpu as pltpu
from jax.experimental.pallas import tpu_sc as plsc

_L = 16
_CHAINS = 4


def _row_argmax(w_v, lane, N):
  nsteps = N // _L
  accs = []
  for a in range(_CHAINS):
    bv = w_v[pl.ds(a * _L, _L)]
    bj = jnp.full((_L,), a, jnp.int32)
    for j in range(a + _CHAINS, nsteps, _CHAINS):
      vals = w_v[pl.ds(j * _L, _L)]
      m = vals > bv
      bv = jnp.where(m, vals, bv)
      bj = jnp.where(m, jnp.int32(j), bj)
    accs.append((bv, bj))
  bv, bj = accs[0]
  for ov, oj in accs[1:]:
    better = (ov > bv) | ((ov == bv) & (oj < bj))
    bv = jnp.where(better, ov, bv)
    bj = jnp.where(better, oj, bj)
  bi = bj * _L + lane
  for k in (8, 4, 2, 1):
    perm = lane ^ k
    ov = bv.at[perm].get(mode="promise_in_bounds")
    oi = bi.at[perm].get(mode="promise_in_bounds")
    better = (ov > bv) | ((ov == bv) & (oi < bi))
    bv = jnp.where(better, ov, bv)
    bi = jnp.where(better, oi, bi)
  return bv[0], bi[0]


def _scale_row(row_v, scale, D):
  for u in range(D // _L):
    row_v[pl.ds(u * _L, _L)] = row_v[pl.ds(u * _L, _L)] * scale


def _select_body(N, D, sw_hbm, mem_hbm, out_hbm,
                 w0, w1, r0, r1, s0, s1, s2, s3):
  cid = lax.axis_index("c")
  sid = lax.axis_index("s")
  wid = sid * 2 + cid
  b0 = wid * 2
  b1 = b0 + 1
  lane = lax.iota(jnp.int32, _L)

  cw0 = pltpu.async_copy(sw_hbm.at[b0], w0, s0)
  cw1 = pltpu.async_copy(sw_hbm.at[b1], w1, s1)

  cw0.wait()
  v0, i0 = _row_argmax(w0, lane, N)
  cr0 = pltpu.async_copy(mem_hbm.at[b0 * N + i0], r0, s2)

  cw1.wait()
  v1, i1 = _row_argmax(w1, lane, N)
  cr1 = pltpu.async_copy(mem_hbm.at[b1 * N + i1], r1, s3)

  one = jnp.float32(1.0)
  sc0 = v0 + (one - v0)
  sc1 = v1 + (one - v1)

  cr0.wait()
  pl.when(sc0 != one)(lambda: _scale_row(r0, sc0, D))
  co0 = pltpu.async_copy(r0, out_hbm.at[b0], s0)

  cr1.wait()
  pl.when(sc1 != one)(lambda: _scale_row(r1, sc1, D))
  co1 = pltpu.async_copy(r1, out_hbm.at[b1], s1)

  co0.wait()
  co1.wait()


def kernel(previous_encoded_m, sim_weights):
  B, N = sim_weights.shape
  D = previous_encoded_m.shape[2]

  table = previous_encoded_m.reshape(B * N, D)

  mesh = plsc.VectorSubcoreMesh(core_axis_name="c", subcore_axis_name="s")
  run = pl.kernel(
      functools.partial(_select_body, N, D),
      mesh=mesh,
      out_type=jax.ShapeDtypeStruct((B, D), jnp.float32),
      scratch_types=[
          pltpu.VMEM((N,), jnp.float32),
          pltpu.VMEM((N,), jnp.float32),
          pltpu.VMEM((D,), jnp.float32),
          pltpu.VMEM((D,), jnp.float32),
          pltpu.SemaphoreType.DMA,
          pltpu.SemaphoreType.DMA,
          pltpu.SemaphoreType.DMA,
          pltpu.SemaphoreType.DMA,
      ],
  )
  return run(sim_weights, table)


# speedup vs baseline: 7.6439x; 1.0405x over previous
import functools

import jax
import jax.numpy as jnp
from jax import lax
from jax.experimental import pallas as pl
from jax.experimental.pallas import tpu as pltpu
from jax.experimental.pallas import tpu_sc as plsc

_L = 16
_CHAINS = 4


def _row_argmax(w_v, lane, N):
  nsteps = N // _L
  def sweep(o, carry):
    accs = list(carry)
    base = o * (2 * _CHAINS)
    for u in range(2 * _CHAINS):
      a = u % _CHAINS
      j = base + u
      bv, bj = accs[a]
      vals = w_v[pl.ds(j * _L, _L)]
      m = vals > bv
      bv = jnp.where(m, vals, bv)
      bj = jnp.where(m, j, bj)
      accs[a] = (bv, bj)
    return tuple(accs)

  init = tuple(
      (jnp.full((_L,), -jnp.inf, jnp.float32), jnp.zeros((_L,), jnp.int32))
      for _ in range(_CHAINS))
  accs = lax.fori_loop(0, nsteps // (2 * _CHAINS), sweep, init)
  bv, bj = accs[0]
  for ov, oj in accs[1:]:
    better = (ov > bv) | ((ov == bv) & (oj < bj))
    bv = jnp.where(better, ov, bv)
    bj = jnp.where(better, oj, bj)
  bi = bj * _L + lane
  for k in (8, 4, 2, 1):
    perm = lane ^ k
    ov = bv.at[perm].get(mode="promise_in_bounds")
    oi = bi.at[perm].get(mode="promise_in_bounds")
    better = (ov > bv) | ((ov == bv) & (oi < bi))
    bv = jnp.where(better, ov, bv)
    bi = jnp.where(better, oi, bi)
  return bv[0], bi[0]


def _scale_row(row_v, scale, D):
  def body(o, _):
    for u in range(4):
      idx = pl.ds((o * 4 + u) * _L, _L)
      row_v[idx] = row_v[idx] * scale
    return 0

  lax.fori_loop(0, D // (4 * _L), body, 0)


def _select_body(N, D, sw_hbm, mem_hbm, out_hbm,
                 w0, w1, r0, r1, s0, s1, s2, s3):
  cid = lax.axis_index("c")
  sid = lax.axis_index("s")
  wid = sid * 2 + cid
  b0 = wid * 2
  b1 = b0 + 1
  lane = lax.iota(jnp.int32, _L)

  cw0 = pltpu.async_copy(sw_hbm.at[b0], w0, s0)
  cw1 = pltpu.async_copy(sw_hbm.at[b1], w1, s1)

  cw0.wait()
  v0, i0 = _row_argmax(w0, lane, N)
  cr0 = pltpu.async_copy(mem_hbm.at[b0 * N + i0], r0, s2)

  cw1.wait()
  v1, i1 = _row_argmax(w1, lane, N)
  cr1 = pltpu.async_copy(mem_hbm.at[b1 * N + i1], r1, s3)

  one = jnp.float32(1.0)
  sc0 = v0 + (one - v0)
  sc1 = v1 + (one - v1)

  cr0.wait()
  pl.when(sc0 != one)(lambda: _scale_row(r0, sc0, D))
  co0 = pltpu.async_copy(r0, out_hbm.at[b0], s0)

  cr1.wait()
  pl.when(sc1 != one)(lambda: _scale_row(r1, sc1, D))
  co1 = pltpu.async_copy(r1, out_hbm.at[b1], s1)

  co0.wait()
  co1.wait()


def kernel(previous_encoded_m, sim_weights):
  B, N = sim_weights.shape
  D = previous_encoded_m.shape[2]

  table = previous_encoded_m.reshape(B * N, D)

  mesh = plsc.VectorSubcoreMesh(core_axis_name="c", subcore_axis_name="s")
  run = pl.kernel(
      functools.partial(_select_body, N, D),
      mesh=mesh,
      out_type=jax.ShapeDtypeStruct((B, D), jnp.float32),
      scratch_types=[
          pltpu.VMEM((N,), jnp.float32),
          pltpu.VMEM((N,), jnp.float32),
          pltpu.VMEM((D,), jnp.float32),
          pltpu.VMEM((D,), jnp.float32),
          pltpu.SemaphoreType.DMA,
          pltpu.SemaphoreType.DMA,
          pltpu.SemaphoreType.DMA,
          pltpu.SemaphoreType.DMA,
      ],
  )
  return run(sim_weights, table)


# speedup vs baseline: 7.7006x; 1.0074x over previous
import functools

import jax
import jax.numpy as jnp
from jax import lax
from jax.experimental import pallas as pl
from jax.experimental.pallas import tpu as pltpu
from jax.experimental.pallas import tpu_sc as plsc

_L = 16
_CHAINS = 2


def _row_argmax(w_v, lane, N):
  nsteps = N // _L
  def sweep(o, carry):
    accs = list(carry)
    base = o * (2 * _CHAINS)
    for u in range(2 * _CHAINS):
      a = u % _CHAINS
      j = base + u
      bv, bj = accs[a]
      vals = w_v[pl.ds(j * _L, _L)]
      m = vals > bv
      bv = jnp.where(m, vals, bv)
      bj = jnp.where(m, j, bj)
      accs[a] = (bv, bj)
    return tuple(accs)

  init = tuple(
      (jnp.full((_L,), -jnp.inf, jnp.float32), jnp.zeros((_L,), jnp.int32))
      for _ in range(_CHAINS))
  accs = lax.fori_loop(0, nsteps // (2 * _CHAINS), sweep, init)
  bv, bj = accs[0]
  for ov, oj in accs[1:]:
    better = (ov > bv) | ((ov == bv) & (oj < bj))
    bv = jnp.where(better, ov, bv)
    bj = jnp.where(better, oj, bj)
  bi = bj * _L + lane
  for k in (8, 4, 2, 1):
    perm = lane ^ k
    ov = bv.at[perm].get(mode="promise_in_bounds")
    oi = bi.at[perm].get(mode="promise_in_bounds")
    better = (ov > bv) | ((ov == bv) & (oi < bi))
    bv = jnp.where(better, ov, bv)
    bi = jnp.where(better, oi, bi)
  return bv[0], bi[0]


def _scale_row(row_v, scale, D):
  def body(o, _):
    for u in range(4):
      idx = pl.ds((o * 4 + u) * _L, _L)
      row_v[idx] = row_v[idx] * scale
    return 0

  lax.fori_loop(0, D // (4 * _L), body, 0)


def _select_body(N, D, sw_hbm, mem_hbm, out_hbm,
                 w0, w1, r0, r1, s0, s1, s2, s3):
  cid = lax.axis_index("c")
  sid = lax.axis_index("s")
  wid = sid * 2 + cid
  b0 = wid * 2
  b1 = b0 + 1
  lane = lax.iota(jnp.int32, _L)

  cw0 = pltpu.async_copy(sw_hbm.at[b0], w0, s0)
  cw1 = pltpu.async_copy(sw_hbm.at[b1], w1, s1)

  cw0.wait()
  v0, i0 = _row_argmax(w0, lane, N)
  cr0 = pltpu.async_copy(mem_hbm.at[b0 * N + i0], r0, s2)

  cw1.wait()
  v1, i1 = _row_argmax(w1, lane, N)
  cr1 = pltpu.async_copy(mem_hbm.at[b1 * N + i1], r1, s3)

  one = jnp.float32(1.0)
  sc0 = v0 + (one - v0)
  sc1 = v1 + (one - v1)

  cr0.wait()
  pl.when(sc0 != one)(lambda: _scale_row(r0, sc0, D))
  co0 = pltpu.async_copy(r0, out_hbm.at[b0], s0)

  cr1.wait()
  pl.when(sc1 != one)(lambda: _scale_row(r1, sc1, D))
  co1 = pltpu.async_copy(r1, out_hbm.at[b1], s1)

  co0.wait()
  co1.wait()


def kernel(previous_encoded_m, sim_weights):
  B, N = sim_weights.shape
  D = previous_encoded_m.shape[2]

  table = previous_encoded_m.reshape(B * N, D)

  mesh = plsc.VectorSubcoreMesh(core_axis_name="c", subcore_axis_name="s")
  run = pl.kernel(
      functools.partial(_select_body, N, D),
      mesh=mesh,
      out_type=jax.ShapeDtypeStruct((B, D), jnp.float32),
      scratch_types=[
          pltpu.VMEM((N,), jnp.float32),
          pltpu.VMEM((N,), jnp.float32),
          pltpu.VMEM((D,), jnp.float32),
          pltpu.VMEM((D,), jnp.float32),
          pltpu.SemaphoreType.DMA,
          pltpu.SemaphoreType.DMA,
          pltpu.SemaphoreType.DMA,
          pltpu.SemaphoreType.DMA,
      ],
  )
  return run(sim_weights, table)
